# Initial kernel scaffold; baseline (speedup 1.0000x reference)
import jax
import jax.numpy as jnp
from jax import lax
from jax.experimental import pallas as pl
from jax.experimental.pallas import tpu as pltpu
from jax.experimental.pallas import tpu_sc as plsc

N = 10000
E = 160000
F = 256
HALF = 128
NC = 2
NS = 16
NW = NC * NS

NP = 10240
EP = 163840
EROWS = EP // 128
RPT = EROWS // NS
ZPT = NP // NS
OPT = N // NS
DPT = NP // NW
NB = 4

BN = 1000
NBLK = N // BN

_mesh = plsc.VectorSubcoreMesh(core_axis_name="c", subcore_axis_name="s")


def _deg_dinv_body(dst2d, zeros1d, ones_h, dinv_out, acc, idxb, onesb, degb, dinvb):
    c = lax.axis_index("c")
    s = lax.axis_index("s")
    wid = c * NS + s

    pltpu.sync_copy(zeros1d, acc.at[pl.ds(s * ZPT, ZPT)])
    pltpu.sync_copy(ones_h, onesb)
    pltpu.sync_copy(dst2d.at[pl.ds(s * RPT, RPT)], idxb)
    plsc.subcore_barrier()

    @pl.loop(0, RPT)
    def _(g):
        pltpu.sync_copy(onesb, acc.at[idxb.at[g]], add=True)

    plsc.subcore_barrier()

    pltpu.sync_copy(acc.at[pl.ds(wid * DPT, DPT)], degb)

    @pl.loop(0, DPT // 16)
    def _(j):
        d = jnp.maximum(degb[pl.ds(j * 16, 16)], 1.0)
        i = lax.bitcast_convert_type(d, jnp.int32)
        y = lax.bitcast_convert_type(jnp.int32(0x5F3759DF) - (i >> 1), jnp.float32)
        y = y * (1.5 - 0.5 * d * y * y)
        y = y * (1.5 - 0.5 * d * y * y)
        y = y * (1.5 - 0.5 * d * y * y)
        dinvb[pl.ds(j * 16, 16)] = y

    pltpu.sync_copy(dinvb, dinv_out.at[pl.ds(wid * DPT, DPT)])


_deg_dinv = pl.kernel(
    _deg_dinv_body,
    out_type=jax.ShapeDtypeStruct((NP,), jnp.float32),
    mesh=_mesh,
    scratch_types=[
        pltpu.VMEM_SHARED((NP,), jnp.float32),
        pltpu.VMEM((RPT, 128), jnp.int32),
        pltpu.VMEM((128,), jnp.float32),
        pltpu.VMEM((DPT,), jnp.float32),
        pltpu.VMEM((DPT,), jnp.float32),
    ],
)


def _agg_body(S, src2d, dst2d, zeros2d, out, acc, srcb, dstb, rb0, rb1, rb2, rb3,
              gsem, ssem):
    c = lax.axis_index("c")
    s = lax.axis_index("s")
    bufs = [rb0, rb1, rb2, rb3]

    pltpu.sync_copy(zeros2d, acc.at[pl.ds(s * ZPT, ZPT)])
    pltpu.sync_copy(src2d.at[pl.ds(s * RPT, RPT)], srcb)
    pltpu.sync_copy(dst2d.at[pl.ds(s * RPT, RPT)], dstb)

    off = c * N

    @pl.loop(0, RPT)
    def _(i):
        for j in range(8):
            srcb[i, pl.ds(j * 16, 16)] = srcb[i, pl.ds(j * 16, 16)] + off

    plsc.subcore_barrier()

    for b in range(NB - 1):
        pltpu.async_copy(S.at[srcb.at[b]], bufs[b], gsem)

    @pl.loop(0, RPT // NB)
    def _(t):
        g0 = t * NB
        for b in range(NB):
            g = g0 + b

            @pl.when(g > 0)
            def _():
                pltpu.make_async_copy(
                    bufs[(b - 1) % NB], acc.at[pl.ds(0, 128)], ssem
                ).wait()

            @pl.when(g + NB - 1 < RPT)
            def _():
                pltpu.async_copy(
                    S.at[srcb.at[g + NB - 1]], bufs[(b - 1) % NB], gsem
                )

            pltpu.make_async_copy(S.at[srcb.at[g]], bufs[b], gsem).wait()
            pltpu.async_copy(bufs[b], acc.at[dstb.at[g]], ssem, add=True)

    pltpu.make_async_copy(
        bufs[NB - 1], acc.at[pl.ds(0, 128)], ssem
    ).wait()

    plsc.subcore_barrier()
    pltpu.sync_copy(
        acc.at[pl.ds(s * OPT, OPT)], out.at[pl.ds(c * N + s * OPT, OPT)]
    )


_agg = pl.kernel(
    _agg_body,
    out_type=jax.ShapeDtypeStruct((2 * N, HALF), jnp.float32),
    mesh=_mesh,
    scratch_types=[
        pltpu.VMEM_SHARED((NP, HALF), jnp.float32),
        pltpu.VMEM((RPT, 128), jnp.int32),
        pltpu.VMEM((RPT, 128), jnp.int32),
        pltpu.VMEM((128, HALF), jnp.float32),
        pltpu.VMEM((128, HALF), jnp.float32),
        pltpu.VMEM((128, HALF), jnp.float32),
        pltpu.VMEM((128, HALF), jnp.float32),
        pltpu.SemaphoreType.DMA,
        pltpu.SemaphoreType.DMA,
    ],
)


def _tc1_body(x_ref, w_ref, dinv_ref, o_ref):
    o_ref[...] = (
        jnp.dot(x_ref[...], w_ref[...], preferred_element_type=jnp.float32)
        * dinv_ref[...]
    )


_tc1 = pl.pallas_call(
    _tc1_body,
    grid=(NBLK, 2),
    in_specs=[
        pl.BlockSpec((BN, F), lambda n, h: (n, 0)),
        pl.BlockSpec((F, HALF), lambda n, h: (0, h)),
        pl.BlockSpec((BN, 1), lambda n, h: (n, 0)),
    ],
    out_specs=pl.BlockSpec((BN, HALF), lambda n, h: (h * NBLK + n, 0)),
    out_shape=jax.ShapeDtypeStruct((2 * N, HALF), jnp.float32),
)


def _tc2_body(alo_ref, ahi_ref, dinv_ref, b1_ref, w2_ref, o_ref):
    a = jnp.concatenate([alo_ref[...], ahi_ref[...]], axis=1)
    hid = jnp.maximum(a * dinv_ref[...] + b1_ref[...], 0.0)
    o_ref[...] = (
        jnp.dot(hid, w2_ref[...], preferred_element_type=jnp.float32)
        * dinv_ref[...]
    )


_tc2 = pl.pallas_call(
    _tc2_body,
    grid=(NBLK, 2),
    in_specs=[
        pl.BlockSpec((BN, HALF), lambda n, h: (n, 0)),
        pl.BlockSpec((BN, HALF), lambda n, h: (NBLK + n, 0)),
        pl.BlockSpec((BN, 1), lambda n, h: (n, 0)),
        pl.BlockSpec((1, F), lambda n, h: (0, 0)),
        pl.BlockSpec((F, HALF), lambda n, h: (0, h)),
    ],
    out_specs=pl.BlockSpec((BN, HALF), lambda n, h: (h * NBLK + n, 0)),
    out_shape=jax.ShapeDtypeStruct((2 * N, HALF), jnp.float32),
)


def _tc3_body(alo_ref, ahi_ref, dinv_ref, b2_ref, o_ref):
    a = jnp.concatenate([alo_ref[...], ahi_ref[...]], axis=1)
    o_ref[...] = a * dinv_ref[...] + b2_ref[...]


_tc3 = pl.pallas_call(
    _tc3_body,
    grid=(NBLK,),
    in_specs=[
        pl.BlockSpec((BN, HALF), lambda n: (n, 0)),
        pl.BlockSpec((BN, HALF), lambda n: (NBLK + n, 0)),
        pl.BlockSpec((BN, 1), lambda n: (n, 0)),
        pl.BlockSpec((1, F), lambda n: (0, 0)),
    ],
    out_specs=pl.BlockSpec((BN, F), lambda n: (n, 0)),
    out_shape=jax.ShapeDtypeStruct((N, F), jnp.float32),
)


def kernel(x, edge_index, W1, b1, W2, b2):
    src = edge_index[0]
    dst = edge_index[1]

    npad = EP - E
    padi = jnp.arange(npad, dtype=jnp.int32)
    src_p = jnp.concatenate([src, padi % N]).reshape(EROWS, 128)
    dst_p = jnp.concatenate([dst, N + padi % (NP - N)]).reshape(EROWS, 128)

    zeros1d = jnp.zeros((ZPT,), jnp.float32)
    zeros2d = jnp.zeros((ZPT, HALF), jnp.float32)
    ones_h = jnp.ones((128,), jnp.float32)

    dinv_pad = _deg_dinv(dst_p, zeros1d, ones_h)
    dinv2d = dinv_pad[:N].reshape(N, 1)

    s1 = _tc1(x, W1, dinv2d)
    a1 = _agg(s1, src_p, dst_p, zeros2d)
    s2 = _tc2(a1, a1, dinv2d, b1.reshape(1, F), W2)
    a2 = _agg(s2, src_p, dst_p, zeros2d)
    return _tc3(a2, a2, dinv2d, b2.reshape(1, F))


# speedup vs baseline: 12.2160x; 12.2160x over previous
import jax
import jax.numpy as jnp
from jax import lax
from jax.experimental import pallas as pl
from jax.experimental.pallas import tpu as pltpu
from jax.experimental.pallas import tpu_sc as plsc

N = 10000
E = 160000
F = 256
HALF = 128
NC = 2
NS = 16
NW = NC * NS

NP = 10240
EP = 163840
EROWS = EP // 128
RPT = EROWS // NS
ZPT = NP // NS
OPT = N // NS
DPT = NP // NW
NB = 2
CH = 40

BN = 1000
NBLK = N // BN

_mesh = plsc.VectorSubcoreMesh(core_axis_name="c", subcore_axis_name="s")


def _deg_dinv_body(dst2d, zeros1d, ones_h, dinv_out, acc, idxb, onesb, degb, dinvb):
    c = lax.axis_index("c")
    s = lax.axis_index("s")
    wid = c * NS + s

    pltpu.sync_copy(zeros1d, acc.at[pl.ds(s * ZPT, ZPT)])
    pltpu.sync_copy(ones_h, onesb)
    pltpu.sync_copy(dst2d.at[pl.ds(s * RPT, RPT)], idxb)
    plsc.subcore_barrier()

    @pl.loop(0, RPT)
    def _(g):
        pltpu.sync_copy(onesb, acc.at[idxb.at[g]], add=True)

    plsc.subcore_barrier()

    pltpu.sync_copy(acc.at[pl.ds(wid * DPT, DPT)], degb)

    @pl.loop(0, DPT // 16)
    def _(j):
        d = jnp.maximum(degb[pl.ds(j * 16, 16)], 1.0)
        i = lax.bitcast_convert_type(d, jnp.int32)
        y = lax.bitcast_convert_type(jnp.int32(0x5F3759DF) - (i >> 1), jnp.float32)
        y = y * (1.5 - 0.5 * d * y * y)
        y = y * (1.5 - 0.5 * d * y * y)
        y = y * (1.5 - 0.5 * d * y * y)
        dinvb[pl.ds(j * 16, 16)] = y

    pltpu.sync_copy(dinvb, dinv_out.at[pl.ds(wid * DPT, DPT)])


_deg_dinv = pl.kernel(
    _deg_dinv_body,
    out_type=jax.ShapeDtypeStruct((NP,), jnp.float32),
    mesh=_mesh,
    scratch_types=[
        pltpu.VMEM_SHARED((NP,), jnp.float32),
        pltpu.VMEM((RPT, 128), jnp.int32),
        pltpu.VMEM((128,), jnp.float32),
        pltpu.VMEM((DPT,), jnp.float32),
        pltpu.VMEM((DPT,), jnp.float32),
    ],
)


def _agg_body(S, src2d, dst2d, zeros2d, out_lo, out_hi, acc, srcb, dstb,
              rb0, rb1, gsem, ssem):
    c = lax.axis_index("c")
    s = lax.axis_index("s")
    bufs = [rb0, rb1]

    pltpu.sync_copy(zeros2d, acc.at[pl.ds(s * ZPT, ZPT)])

    off = c * NP

    plsc.subcore_barrier()

    @pl.loop(0, RPT // CH)
    def _(ch):
        base = s * RPT + ch * CH
        pltpu.sync_copy(src2d.at[pl.ds(base, CH)], srcb)
        pltpu.sync_copy(dst2d.at[pl.ds(base, CH)], dstb)

        @pl.loop(0, CH)
        def _(i):
            for j in range(8):
                srcb[i, pl.ds(j * 16, 16)] = srcb[i, pl.ds(j * 16, 16)] + off

        pltpu.async_copy(S.at[srcb.at[0]], bufs[0], gsem)

        @pl.loop(0, CH // NB)
        def _(t):
            for b in range(NB):
                r = t * NB + b
                pltpu.make_async_copy(S.at[srcb.at[r]], bufs[b], gsem).wait()

                @pl.when(r >= 1)
                def _():
                    pltpu.make_async_copy(
                        bufs[1 - b], acc.at[pl.ds(0, 128)], ssem
                    ).wait()

                @pl.when(r + 1 < CH)
                def _():
                    pltpu.async_copy(S.at[srcb.at[r + 1]], bufs[1 - b], gsem)

                pltpu.async_copy(bufs[b], acc.at[dstb.at[r]], ssem, add=True)

        pltpu.make_async_copy(
            bufs[1], acc.at[pl.ds(0, 128)], ssem
        ).wait()

    plsc.subcore_barrier()

    @pl.when(c == 0)
    def _():
        pltpu.sync_copy(acc.at[pl.ds(s * ZPT, ZPT)], out_lo.at[pl.ds(s * ZPT, ZPT)])

    @pl.when(c == 1)
    def _():
        pltpu.sync_copy(acc.at[pl.ds(s * ZPT, ZPT)], out_hi.at[pl.ds(s * ZPT, ZPT)])


_agg = pl.kernel(
    _agg_body,
    out_type=(
        jax.ShapeDtypeStruct((NP, HALF), jnp.float32),
        jax.ShapeDtypeStruct((NP, HALF), jnp.float32),
    ),
    mesh=_mesh,
    scratch_types=[
        pltpu.VMEM_SHARED((NP, HALF), jnp.float32),
        pltpu.VMEM((CH, 128), jnp.int32),
        pltpu.VMEM((CH, 128), jnp.int32),
        pltpu.VMEM((128, HALF), jnp.float32),
        pltpu.VMEM((128, HALF), jnp.float32),
        pltpu.SemaphoreType.DMA,
        pltpu.SemaphoreType.DMA,
    ],
)


def _tc1_body(x_ref, w_ref, dinv_ref, o_ref):
    res = (
        jnp.dot(x_ref[...], w_ref[...], preferred_element_type=jnp.float32)
        * dinv_ref[...]
    )
    o_ref[...] = res[None, :, :]


_tc1 = pl.pallas_call(
    _tc1_body,
    grid=(NBLK, 2),
    in_specs=[
        pl.BlockSpec((BN, F), lambda n, h: (n, 0)),
        pl.BlockSpec((F, HALF), lambda n, h: (0, h)),
        pl.BlockSpec((BN, 1), lambda n, h: (n, 0)),
    ],
    out_specs=pl.BlockSpec((1, BN, HALF), lambda n, h: (h, n, 0)),
    out_shape=jax.ShapeDtypeStruct((2, NP, HALF), jnp.float32),
)


def _tc2_body(alo_ref, ahi_ref, dinv_ref, b1_ref, w2_ref, o_ref):
    a = jnp.concatenate([alo_ref[...], ahi_ref[...]], axis=1)
    hid = jnp.maximum(a * dinv_ref[...] + b1_ref[...], 0.0)
    res = (
        jnp.dot(hid, w2_ref[...], preferred_element_type=jnp.float32)
        * dinv_ref[...]
    )
    o_ref[...] = res[None, :, :]


_tc2 = pl.pallas_call(
    _tc2_body,
    grid=(NBLK, 2),
    in_specs=[
        pl.BlockSpec((BN, HALF), lambda n, h: (n, 0)),
        pl.BlockSpec((BN, HALF), lambda n, h: (n, 0)),
        pl.BlockSpec((BN, 1), lambda n, h: (n, 0)),
        pl.BlockSpec((1, F), lambda n, h: (0, 0)),
        pl.BlockSpec((F, HALF), lambda n, h: (0, h)),
    ],
    out_specs=pl.BlockSpec((1, BN, HALF), lambda n, h: (h, n, 0)),
    out_shape=jax.ShapeDtypeStruct((2, NP, HALF), jnp.float32),
)


def _tc3_body(alo_ref, ahi_ref, dinv_ref, b2_ref, o_ref):
    a = jnp.concatenate([alo_ref[...], ahi_ref[...]], axis=1)
    o_ref[...] = a * dinv_ref[...] + b2_ref[...]


_tc3 = pl.pallas_call(
    _tc3_body,
    grid=(NBLK,),
    in_specs=[
        pl.BlockSpec((BN, HALF), lambda n: (n, 0)),
        pl.BlockSpec((BN, HALF), lambda n: (n, 0)),
        pl.BlockSpec((BN, 1), lambda n: (n, 0)),
        pl.BlockSpec((1, F), lambda n: (0, 0)),
    ],
    out_specs=pl.BlockSpec((BN, F), lambda n: (n, 0)),
    out_shape=jax.ShapeDtypeStruct((N, F), jnp.float32),
)


def kernel(x, edge_index, W1, b1, W2, b2):
    src = edge_index[0]
    dst = edge_index[1]

    npad = EP - E
    padi = jnp.arange(npad, dtype=jnp.int32)
    src_p = jnp.concatenate([src, padi % N]).reshape(EROWS, 128)
    dst_p = jnp.concatenate([dst, N + padi % (NP - N)]).reshape(EROWS, 128)

    zeros1d = jnp.zeros((ZPT,), jnp.float32)
    zeros2d = jnp.zeros((ZPT, HALF), jnp.float32)
    ones_h = jnp.ones((128,), jnp.float32)

    dinv_pad = _deg_dinv(dst_p, zeros1d, ones_h)
    dinv2d = dinv_pad[:N].reshape(N, 1)

    s1 = _tc1(x, W1, dinv2d).reshape(2 * NP, HALF)
    a1_lo, a1_hi = _agg(s1, src_p, dst_p, zeros2d)
    s2 = _tc2(a1_lo, a1_hi, dinv2d, b1.reshape(1, F), W2).reshape(2 * NP, HALF)
    a2_lo, a2_hi = _agg(s2, src_p, dst_p, zeros2d)
    return _tc3(a2_lo, a2_hi, dinv2d, b2.reshape(1, F))


# speedup vs baseline: 13.0844x; 1.0711x over previous
import jax
import jax.numpy as jnp
from jax import lax
from jax.experimental import pallas as pl
from jax.experimental.pallas import tpu as pltpu
from jax.experimental.pallas import tpu_sc as plsc

N = 10000
E = 160000
F = 256
HALF = 128
NC = 2
NS = 16
NW = NC * NS

NP = 10240
EP = 163840
SLOT = 64
EROWS = EP // SLOT
RPT = EROWS // NS
ZPT = NP // NS
DPT = NP // NW
NB = 4
CH = RPT // 4

BN = 1000
NBLK = N // BN

_mesh = plsc.VectorSubcoreMesh(core_axis_name="c", subcore_axis_name="s")


def _deg_dinv_body(dst2d, zeros1d, ones_h, dinv_out, acc, idxb, onesb, degb, dinvb):
    c = lax.axis_index("c")
    s = lax.axis_index("s")
    wid = c * NS + s

    pltpu.sync_copy(zeros1d, acc.at[pl.ds(s * ZPT, ZPT)])
    pltpu.sync_copy(ones_h, onesb)
    pltpu.sync_copy(dst2d.at[pl.ds(s * RPT, RPT)], idxb)
    plsc.subcore_barrier()

    @pl.loop(0, RPT, step=8)
    def _(g0):
        for u in range(8):
            pltpu.sync_copy(onesb, acc.at[idxb.at[g0 + u]], add=True)

    plsc.subcore_barrier()

    pltpu.sync_copy(acc.at[pl.ds(wid * DPT, DPT)], degb)

    @pl.loop(0, DPT // 16)
    def _(j):
        d = jnp.maximum(degb[pl.ds(j * 16, 16)], 1.0)
        i = lax.bitcast_convert_type(d, jnp.int32)
        y = lax.bitcast_convert_type(jnp.int32(0x5F3759DF) - (i >> 1), jnp.float32)
        y = y * (1.5 - 0.5 * d * y * y)
        y = y * (1.5 - 0.5 * d * y * y)
        y = y * (1.5 - 0.5 * d * y * y)
        dinvb[pl.ds(j * 16, 16)] = y

    pltpu.sync_copy(dinvb, dinv_out.at[pl.ds(wid * DPT, DPT)])


_deg_dinv = pl.kernel(
    _deg_dinv_body,
    out_type=jax.ShapeDtypeStruct((NP,), jnp.float32),
    mesh=_mesh,
    scratch_types=[
        pltpu.VMEM_SHARED((NP,), jnp.float32),
        pltpu.VMEM((RPT, SLOT), jnp.int32),
        pltpu.VMEM((SLOT,), jnp.float32),
        pltpu.VMEM((DPT,), jnp.float32),
        pltpu.VMEM((DPT,), jnp.float32),
    ],
)


def _agg_body(S, src2d, dst2d, zeros2d, out_lo, out_hi, acc, srcb, dstb,
              rb0, rb1, rb2, rb3, gsem, ssem):
    c = lax.axis_index("c")
    s = lax.axis_index("s")
    bufs = [rb0, rb1, rb2, rb3]

    def wait_scatter(b):
        pltpu.make_async_copy(bufs[b], acc.at[pl.ds(0, SLOT)], ssem).wait()

    pltpu.sync_copy(zeros2d, acc.at[pl.ds(s * ZPT, ZPT)])

    off = c * NP

    plsc.subcore_barrier()

    @pl.loop(0, RPT // CH)
    def _(ch):
        base = s * RPT + ch * CH
        pltpu.sync_copy(src2d.at[pl.ds(base, CH)], srcb)
        pltpu.sync_copy(dst2d.at[pl.ds(base, CH)], dstb)

        @pl.loop(0, CH)
        def _(i):
            for j in range(SLOT // 16):
                srcb[i, pl.ds(j * 16, 16)] = srcb[i, pl.ds(j * 16, 16)] + off

        pltpu.async_copy(S.at[srcb.at[0]], bufs[0], gsem)
        pltpu.async_copy(S.at[srcb.at[1]], bufs[1], gsem)

        @pl.loop(0, CH // NB)
        def _(t):
            for b in range(NB):
                r = t * NB + b
                pltpu.make_async_copy(S.at[srcb.at[r]], bufs[b], gsem).wait()
                pltpu.async_copy(bufs[b], acc.at[dstb.at[r]], ssem, add=True)

                @pl.when(r >= 2)
                def _():
                    wait_scatter((b + 2) % NB)

                @pl.when(r + 2 < CH)
                def _():
                    pltpu.async_copy(
                        S.at[srcb.at[r + 2]], bufs[(b + 2) % NB], gsem
                    )

        wait_scatter(NB - 2)
        wait_scatter(NB - 1)

    plsc.subcore_barrier()

    @pl.when(c == 0)
    def _():
        pltpu.sync_copy(acc.at[pl.ds(s * ZPT, ZPT)], out_lo.at[pl.ds(s * ZPT, ZPT)])

    @pl.when(c == 1)
    def _():
        pltpu.sync_copy(acc.at[pl.ds(s * ZPT, ZPT)], out_hi.at[pl.ds(s * ZPT, ZPT)])


_agg = pl.kernel(
    _agg_body,
    out_type=(
        jax.ShapeDtypeStruct((NP, HALF), jnp.float32),
        jax.ShapeDtypeStruct((NP, HALF), jnp.float32),
    ),
    mesh=_mesh,
    scratch_types=[
        pltpu.VMEM_SHARED((NP, HALF), jnp.float32),
        pltpu.VMEM((CH, SLOT), jnp.int32),
        pltpu.VMEM((CH, SLOT), jnp.int32),
        pltpu.VMEM((SLOT, HALF), jnp.float32),
        pltpu.VMEM((SLOT, HALF), jnp.float32),
        pltpu.VMEM((SLOT, HALF), jnp.float32),
        pltpu.VMEM((SLOT, HALF), jnp.float32),
        pltpu.SemaphoreType.DMA,
        pltpu.SemaphoreType.DMA,
    ],
)


def _tc1_body(x_ref, w_ref, dinv_ref, o_ref):
    res = (
        jnp.dot(x_ref[...], w_ref[...], preferred_element_type=jnp.float32)
        * dinv_ref[...]
    )
    o_ref[...] = res[None, :, :]


_tc1 = pl.pallas_call(
    _tc1_body,
    grid=(NBLK, 2),
    in_specs=[
        pl.BlockSpec((BN, F), lambda n, h: (n, 0)),
        pl.BlockSpec((F, HALF), lambda n, h: (0, h)),
        pl.BlockSpec((BN, 1), lambda n, h: (n, 0)),
    ],
    out_specs=pl.BlockSpec((1, BN, HALF), lambda n, h: (h, n, 0)),
    out_shape=jax.ShapeDtypeStruct((2, NP, HALF), jnp.float32),
)


def _tc2_body(alo_ref, ahi_ref, dinv_ref, b1_ref, w2_ref, o_ref):
    a = jnp.concatenate([alo_ref[...], ahi_ref[...]], axis=1)
    hid = jnp.maximum(a * dinv_ref[...] + b1_ref[...], 0.0)
    res = (
        jnp.dot(hid, w2_ref[...], preferred_element_type=jnp.float32)
        * dinv_ref[...]
    )
    o_ref[...] = res[None, :, :]


_tc2 = pl.pallas_call(
    _tc2_body,
    grid=(NBLK, 2),
    in_specs=[
        pl.BlockSpec((BN, HALF), lambda n, h: (n, 0)),
        pl.BlockSpec((BN, HALF), lambda n, h: (n, 0)),
        pl.BlockSpec((BN, 1), lambda n, h: (n, 0)),
        pl.BlockSpec((1, F), lambda n, h: (0, 0)),
        pl.BlockSpec((F, HALF), lambda n, h: (0, h)),
    ],
    out_specs=pl.BlockSpec((1, BN, HALF), lambda n, h: (h, n, 0)),
    out_shape=jax.ShapeDtypeStruct((2, NP, HALF), jnp.float32),
)


def _tc3_body(alo_ref, ahi_ref, dinv_ref, b2_ref, o_ref):
    a = jnp.concatenate([alo_ref[...], ahi_ref[...]], axis=1)
    o_ref[...] = a * dinv_ref[...] + b2_ref[...]


_tc3 = pl.pallas_call(
    _tc3_body,
    grid=(NBLK,),
    in_specs=[
        pl.BlockSpec((BN, HALF), lambda n: (n, 0)),
        pl.BlockSpec((BN, HALF), lambda n: (n, 0)),
        pl.BlockSpec((BN, 1), lambda n: (n, 0)),
        pl.BlockSpec((1, F), lambda n: (0, 0)),
    ],
    out_specs=pl.BlockSpec((BN, F), lambda n: (n, 0)),
    out_shape=jax.ShapeDtypeStruct((N, F), jnp.float32),
)


def kernel(x, edge_index, W1, b1, W2, b2):
    src = edge_index[0]
    dst = edge_index[1]

    npad = EP - E
    padi = jnp.arange(npad, dtype=jnp.int32)
    src_p = jnp.concatenate([src, padi % N]).reshape(EROWS, SLOT)
    dst_p = jnp.concatenate([dst, N + padi % (NP - N)]).reshape(EROWS, SLOT)

    zeros1d = jnp.zeros((ZPT,), jnp.float32)
    zeros2d = jnp.zeros((ZPT, HALF), jnp.float32)
    ones_h = jnp.ones((SLOT,), jnp.float32)

    dinv_pad = _deg_dinv(dst_p, zeros1d, ones_h)
    dinv2d = dinv_pad[:N].reshape(N, 1)

    s1 = _tc1(x, W1, dinv2d).reshape(2 * NP, HALF)
    a1_lo, a1_hi = _agg(s1, src_p, dst_p, zeros2d)
    s2 = _tc2(a1_lo, a1_hi, dinv2d, b1.reshape(1, F), W2).reshape(2 * NP, HALF)
    a2_lo, a2_hi = _agg(s2, src_p, dst_p, zeros2d)
    return _tc3(a2_lo, a2_hi, dinv2d, b2.reshape(1, F))


# speedup vs baseline: 13.1321x; 1.0036x over previous
import jax
import jax.numpy as jnp
from jax import lax
from jax.experimental import pallas as pl
from jax.experimental.pallas import tpu as pltpu
from jax.experimental.pallas import tpu_sc as plsc

N = 10000
E = 160000
F = 256
HALF = 128
NC = 2
NS = 16
NW = NC * NS

NP = 10240
EP = 163840
SLOT = 80
EROWS = EP // SLOT
RPT = EROWS // NS
ZPT = NP // NS
DPT = NP // NW
NB = 4
CH = RPT // 8

BN = 1000
NBLK = N // BN

_mesh = plsc.VectorSubcoreMesh(core_axis_name="c", subcore_axis_name="s")


def _deg_dinv_body(dst2d, zeros1d, ones_h, dinv_out, acc, idxb, onesb, degb,
                   dinvb, hsem):
    c = lax.axis_index("c")
    s = lax.axis_index("s")
    wid = c * NS + s

    pltpu.sync_copy(zeros1d, acc.at[pl.ds(s * ZPT, ZPT)])
    pltpu.sync_copy(ones_h, onesb)
    pltpu.sync_copy(dst2d.at[pl.ds(s * RPT, RPT)], idxb)
    plsc.subcore_barrier()

    @pl.loop(0, RPT, step=16)
    def _(g0):
        for u in range(16):
            pltpu.async_copy(onesb, acc.at[idxb.at[g0 + u]], hsem, add=True)
        for u in range(16):
            pltpu.make_async_copy(onesb, acc.at[pl.ds(0, SLOT)], hsem).wait()

    plsc.subcore_barrier()

    pltpu.sync_copy(acc.at[pl.ds(wid * DPT, DPT)], degb)

    @pl.loop(0, DPT // 16)
    def _(j):
        d = jnp.maximum(degb[pl.ds(j * 16, 16)], 1.0)
        i = lax.bitcast_convert_type(d, jnp.int32)
        y = lax.bitcast_convert_type(jnp.int32(0x5F3759DF) - (i >> 1), jnp.float32)
        y = y * (1.5 - 0.5 * d * y * y)
        y = y * (1.5 - 0.5 * d * y * y)
        y = y * (1.5 - 0.5 * d * y * y)
        dinvb[pl.ds(j * 16, 16)] = y

    pltpu.sync_copy(dinvb, dinv_out.at[pl.ds(wid * DPT, DPT)])


_deg_dinv = pl.kernel(
    _deg_dinv_body,
    out_type=jax.ShapeDtypeStruct((NP,), jnp.float32),
    mesh=_mesh,
    scratch_types=[
        pltpu.VMEM_SHARED((NP,), jnp.float32),
        pltpu.VMEM((RPT, SLOT), jnp.int32),
        pltpu.VMEM((SLOT,), jnp.float32),
        pltpu.VMEM((DPT,), jnp.float32),
        pltpu.VMEM((DPT,), jnp.float32),
        pltpu.SemaphoreType.DMA,
    ],
)


def _agg_body(S, src2d, dst2d, zeros2d, out_lo, out_hi, acc, srcb, dstb,
              rb0, rb1, rb2, rb3, gsem, ssem):
    c = lax.axis_index("c")
    s = lax.axis_index("s")
    bufs = [rb0, rb1, rb2, rb3]

    def wait_scatter(b):
        pltpu.make_async_copy(bufs[b], acc.at[pl.ds(0, SLOT)], ssem).wait()

    pltpu.sync_copy(zeros2d, acc.at[pl.ds(s * ZPT, ZPT)])

    off = c * NP

    plsc.subcore_barrier()

    @pl.loop(0, RPT // CH)
    def _(ch):
        base = s * RPT + ch * CH
        pltpu.sync_copy(src2d.at[pl.ds(base, CH)], srcb)
        pltpu.sync_copy(dst2d.at[pl.ds(base, CH)], dstb)

        @pl.loop(0, CH)
        def _(i):
            for j in range(SLOT // 16):
                srcb[i, pl.ds(j * 16, 16)] = srcb[i, pl.ds(j * 16, 16)] + off

        pltpu.async_copy(S.at[srcb.at[0]], bufs[0], gsem)
        pltpu.async_copy(S.at[srcb.at[1]], bufs[1], gsem)

        @pl.loop(0, CH // NB)
        def _(t):
            for b in range(NB):
                r = t * NB + b
                pltpu.make_async_copy(S.at[srcb.at[r]], bufs[b], gsem).wait()
                pltpu.async_copy(bufs[b], acc.at[dstb.at[r]], ssem, add=True)

                @pl.when(r >= 2)
                def _():
                    wait_scatter((b + 2) % NB)

                @pl.when(r + 2 < CH)
                def _():
                    pltpu.async_copy(
                        S.at[srcb.at[r + 2]], bufs[(b + 2) % NB], gsem
                    )

        wait_scatter(NB - 2)
        wait_scatter(NB - 1)

    plsc.subcore_barrier()

    @pl.when(c == 0)
    def _():
        pltpu.sync_copy(acc.at[pl.ds(s * ZPT, ZPT)], out_lo.at[pl.ds(s * ZPT, ZPT)])

    @pl.when(c == 1)
    def _():
        pltpu.sync_copy(acc.at[pl.ds(s * ZPT, ZPT)], out_hi.at[pl.ds(s * ZPT, ZPT)])


_agg = pl.kernel(
    _agg_body,
    out_type=(
        jax.ShapeDtypeStruct((NP, HALF), jnp.float32),
        jax.ShapeDtypeStruct((NP, HALF), jnp.float32),
    ),
    mesh=_mesh,
    scratch_types=[
        pltpu.VMEM_SHARED((NP, HALF), jnp.float32),
        pltpu.VMEM((CH, SLOT), jnp.int32),
        pltpu.VMEM((CH, SLOT), jnp.int32),
        pltpu.VMEM((SLOT, HALF), jnp.float32),
        pltpu.VMEM((SLOT, HALF), jnp.float32),
        pltpu.VMEM((SLOT, HALF), jnp.float32),
        pltpu.VMEM((SLOT, HALF), jnp.float32),
        pltpu.SemaphoreType.DMA,
        pltpu.SemaphoreType.DMA,
    ],
)


def _tc1a_body(x_ref, w_ref, o_ref):
    res = jnp.dot(x_ref[...], w_ref[...], preferred_element_type=jnp.float32)
    o_ref[...] = res[None, :, :]


_tc1a = pl.pallas_call(
    _tc1a_body,
    grid=(NBLK, 2),
    in_specs=[
        pl.BlockSpec((BN, F), lambda n, h: (n, 0)),
        pl.BlockSpec((F, HALF), lambda n, h: (0, h)),
    ],
    out_specs=pl.BlockSpec((1, BN, HALF), lambda n, h: (h, n, 0)),
    out_shape=jax.ShapeDtypeStruct((2, NP, HALF), jnp.float32),
)


def _tc1b_body(s_ref, dinv_ref, o_ref):
    o_ref[...] = s_ref[...] * dinv_ref[...][None]


_tc1b = pl.pallas_call(
    _tc1b_body,
    grid=(NBLK, 2),
    in_specs=[
        pl.BlockSpec((1, BN, HALF), lambda n, h: (h, n, 0)),
        pl.BlockSpec((BN, 1), lambda n, h: (n, 0)),
    ],
    out_specs=pl.BlockSpec((1, BN, HALF), lambda n, h: (h, n, 0)),
    out_shape=jax.ShapeDtypeStruct((2, NP, HALF), jnp.float32),
)


def _tc2_body(alo_ref, ahi_ref, dinv_ref, b1_ref, w2_ref, o_ref):
    a = jnp.concatenate([alo_ref[...], ahi_ref[...]], axis=1)
    hid = jnp.maximum(a * dinv_ref[...] + b1_ref[...], 0.0)
    res = (
        jnp.dot(hid, w2_ref[...], preferred_element_type=jnp.float32)
        * dinv_ref[...]
    )
    o_ref[...] = res[None, :, :]


_tc2 = pl.pallas_call(
    _tc2_body,
    grid=(NBLK, 2),
    in_specs=[
        pl.BlockSpec((BN, HALF), lambda n, h: (n, 0)),
        pl.BlockSpec((BN, HALF), lambda n, h: (n, 0)),
        pl.BlockSpec((BN, 1), lambda n, h: (n, 0)),
        pl.BlockSpec((1, F), lambda n, h: (0, 0)),
        pl.BlockSpec((F, HALF), lambda n, h: (0, h)),
    ],
    out_specs=pl.BlockSpec((1, BN, HALF), lambda n, h: (h, n, 0)),
    out_shape=jax.ShapeDtypeStruct((2, NP, HALF), jnp.float32),
)


def _tc3_body(alo_ref, ahi_ref, dinv_ref, b2_ref, o_ref):
    a = jnp.concatenate([alo_ref[...], ahi_ref[...]], axis=1)
    o_ref[...] = a * dinv_ref[...] + b2_ref[...]


_tc3 = pl.pallas_call(
    _tc3_body,
    grid=(NBLK,),
    in_specs=[
        pl.BlockSpec((BN, HALF), lambda n: (n, 0)),
        pl.BlockSpec((BN, HALF), lambda n: (n, 0)),
        pl.BlockSpec((BN, 1), lambda n: (n, 0)),
        pl.BlockSpec((1, F), lambda n: (0, 0)),
    ],
    out_specs=pl.BlockSpec((BN, F), lambda n: (n, 0)),
    out_shape=jax.ShapeDtypeStruct((N, F), jnp.float32),
)


def kernel(x, edge_index, W1, b1, W2, b2):
    src = edge_index[0]
    dst = edge_index[1]

    npad = EP - E
    padi = jnp.arange(npad, dtype=jnp.int32)
    src_p = jnp.concatenate([src, padi % N]).reshape(EROWS, SLOT)
    dst_p = jnp.concatenate([dst, N + padi % (NP - N)]).reshape(EROWS, SLOT)

    zeros1d = jnp.zeros((ZPT,), jnp.float32)
    zeros2d = jnp.zeros((ZPT, HALF), jnp.float32)
    ones_h = jnp.ones((SLOT,), jnp.float32)

    dinv_pad = _deg_dinv(dst_p, zeros1d, ones_h)
    s1raw = _tc1a(x, W1)
    dinv2d = dinv_pad[:N].reshape(N, 1)

    s1 = _tc1b(s1raw, dinv2d).reshape(2 * NP, HALF)
    a1_lo, a1_hi = _agg(s1, src_p, dst_p, zeros2d)
    s2 = _tc2(a1_lo, a1_hi, dinv2d, b1.reshape(1, F), W2).reshape(2 * NP, HALF)
    a2_lo, a2_hi = _agg(s2, src_p, dst_p, zeros2d)
    return _tc3(a2_lo, a2_hi, dinv2d, b2.reshape(1, F))


# speedup vs baseline: 13.4202x; 1.0219x over previous
import jax
import jax.numpy as jnp
from jax import lax
from jax.experimental import pallas as pl
from jax.experimental.pallas import tpu as pltpu
from jax.experimental.pallas import tpu_sc as plsc

N = 10000
E = 160000
F = 256
HALF = 128
NC = 2
NS = 16
NW = NC * NS

NP = 10240
EP = 163840
SLOT = 80
EROWS = EP // SLOT
RPT = EROWS // NS
ZPT = NP // NS
DPT = NP // NW
NB = 4
CH = RPT // 8

BN = 1024
NBLK = NP // BN
BNO = 1000
NBLKO = N // BNO

_mesh = plsc.VectorSubcoreMesh(core_axis_name="c", subcore_axis_name="s")


def _deg_dinv_body(dst2d, zeros1d, ones_h, dinv_out, acc, idxb, onesb, degb,
                   dinvb, hsem):
    c = lax.axis_index("c")
    s = lax.axis_index("s")
    wid = c * NS + s

    pltpu.sync_copy(zeros1d, acc.at[pl.ds(s * ZPT, ZPT)])
    pltpu.sync_copy(ones_h, onesb)
    pltpu.sync_copy(dst2d.at[pl.ds(s * RPT, RPT)], idxb)
    plsc.subcore_barrier()

    @pl.loop(0, RPT, step=16)
    def _(g0):
        for u in range(16):
            pltpu.async_copy(onesb, acc.at[idxb.at[g0 + u]], hsem, add=True)
        for u in range(16):
            pltpu.make_async_copy(onesb, acc.at[pl.ds(0, SLOT)], hsem).wait()

    plsc.subcore_barrier()

    pltpu.sync_copy(acc.at[pl.ds(wid * DPT, DPT)], degb)

    @pl.loop(0, DPT // 16)
    def _(j):
        d = jnp.maximum(degb[pl.ds(j * 16, 16)], 1.0)
        i = lax.bitcast_convert_type(d, jnp.int32)
        y = lax.bitcast_convert_type(jnp.int32(0x5F3759DF) - (i >> 1), jnp.float32)
        y = y * (1.5 - 0.5 * d * y * y)
        y = y * (1.5 - 0.5 * d * y * y)
        y = y * (1.5 - 0.5 * d * y * y)
        dinvb[pl.ds(j * 16, 16)] = y

    pltpu.sync_copy(dinvb, dinv_out.at[pl.ds(wid * DPT, DPT)])


_deg_dinv = pl.kernel(
    _deg_dinv_body,
    out_type=jax.ShapeDtypeStruct((NP,), jnp.float32),
    mesh=_mesh,
    scratch_types=[
        pltpu.VMEM_SHARED((NP,), jnp.float32),
        pltpu.VMEM((RPT, SLOT), jnp.int32),
        pltpu.VMEM((SLOT,), jnp.float32),
        pltpu.VMEM((DPT,), jnp.float32),
        pltpu.VMEM((DPT,), jnp.float32),
        pltpu.SemaphoreType.DMA,
    ],
)


def _agg_body(S, src2d, dst2d, zeros2d, out_lo, out_hi, acc, srcb, dstb,
              rb0, rb1, rb2, rb3, gsem, ssem):
    c = lax.axis_index("c")
    s = lax.axis_index("s")
    bufs = [rb0, rb1, rb2, rb3]

    def wait_scatter(b):
        pltpu.make_async_copy(bufs[b], acc.at[pl.ds(0, SLOT)], ssem).wait()

    pltpu.sync_copy(zeros2d, acc.at[pl.ds(s * ZPT, ZPT)])

    off = c * NP

    plsc.subcore_barrier()

    @pl.loop(0, RPT // CH)
    def _(ch):
        base = s * RPT + ch * CH
        pltpu.sync_copy(src2d.at[pl.ds(base, CH)], srcb)
        pltpu.sync_copy(dst2d.at[pl.ds(base, CH)], dstb)

        @pl.loop(0, CH)
        def _(i):
            for j in range(SLOT // 16):
                srcb[i, pl.ds(j * 16, 16)] = srcb[i, pl.ds(j * 16, 16)] + off

        pltpu.async_copy(S.at[srcb.at[0]], bufs[0], gsem)
        pltpu.async_copy(S.at[srcb.at[1]], bufs[1], gsem)

        @pl.loop(0, CH // NB)
        def _(t):
            for b in range(NB):
                r = t * NB + b
                pltpu.make_async_copy(S.at[srcb.at[r]], bufs[b], gsem).wait()
                pltpu.async_copy(bufs[b], acc.at[dstb.at[r]], ssem, add=True)

                @pl.when(r >= 2)
                def _():
                    wait_scatter((b + 2) % NB)

                @pl.when(r + 2 < CH)
                def _():
                    pltpu.async_copy(
                        S.at[srcb.at[r + 2]], bufs[(b + 2) % NB], gsem
                    )

        wait_scatter(NB - 2)
        wait_scatter(NB - 1)

    plsc.subcore_barrier()

    @pl.when(c == 0)
    def _():
        pltpu.sync_copy(acc.at[pl.ds(s * ZPT, ZPT)], out_lo.at[pl.ds(s * ZPT, ZPT)])

    @pl.when(c == 1)
    def _():
        pltpu.sync_copy(acc.at[pl.ds(s * ZPT, ZPT)], out_hi.at[pl.ds(s * ZPT, ZPT)])


_agg = pl.kernel(
    _agg_body,
    out_type=(
        jax.ShapeDtypeStruct((NP, HALF), jnp.float32),
        jax.ShapeDtypeStruct((NP, HALF), jnp.float32),
    ),
    mesh=_mesh,
    scratch_types=[
        pltpu.VMEM_SHARED((NP, HALF), jnp.float32),
        pltpu.VMEM((CH, SLOT), jnp.int32),
        pltpu.VMEM((CH, SLOT), jnp.int32),
        pltpu.VMEM((SLOT, HALF), jnp.float32),
        pltpu.VMEM((SLOT, HALF), jnp.float32),
        pltpu.VMEM((SLOT, HALF), jnp.float32),
        pltpu.VMEM((SLOT, HALF), jnp.float32),
        pltpu.SemaphoreType.DMA,
        pltpu.SemaphoreType.DMA,
    ],
)


def _tc1_body(x_ref, w_ref, dinv_ref, o_ref):
    o_ref[...] = (
        jnp.dot(x_ref[...], w_ref[...], preferred_element_type=jnp.float32)
        * dinv_ref[...]
    )


_tc1 = pl.pallas_call(
    _tc1_body,
    grid=(NBLK, 2),
    in_specs=[
        pl.BlockSpec((BN, F), lambda n, h: (n, 0)),
        pl.BlockSpec((F, HALF), lambda n, h: (0, h)),
        pl.BlockSpec((BN, 1), lambda n, h: (n, 0)),
    ],
    out_specs=pl.BlockSpec((BN, HALF), lambda n, h: (h * NBLK + n, 0)),
    out_shape=jax.ShapeDtypeStruct((2 * NP, HALF), jnp.float32),
)


def _tc2_body(alo_ref, ahi_ref, dinv_ref, b1_ref, w2_ref, o_ref):
    a = jnp.concatenate([alo_ref[...], ahi_ref[...]], axis=1)
    hid = jnp.maximum(a * dinv_ref[...] + b1_ref[...], 0.0)
    o_ref[...] = (
        jnp.dot(hid, w2_ref[...], preferred_element_type=jnp.float32)
        * dinv_ref[...]
    )


_tc2 = pl.pallas_call(
    _tc2_body,
    grid=(NBLK, 2),
    in_specs=[
        pl.BlockSpec((BN, HALF), lambda n, h: (n, 0)),
        pl.BlockSpec((BN, HALF), lambda n, h: (n, 0)),
        pl.BlockSpec((BN, 1), lambda n, h: (n, 0)),
        pl.BlockSpec((1, F), lambda n, h: (0, 0)),
        pl.BlockSpec((F, HALF), lambda n, h: (0, h)),
    ],
    out_specs=pl.BlockSpec((BN, HALF), lambda n, h: (h * NBLK + n, 0)),
    out_shape=jax.ShapeDtypeStruct((2 * NP, HALF), jnp.float32),
)


def _tc3_body(alo_ref, ahi_ref, dinv_ref, b2_ref, o_ref):
    a = jnp.concatenate([alo_ref[...], ahi_ref[...]], axis=1)
    o_ref[...] = a * dinv_ref[...] + b2_ref[...]


_tc3 = pl.pallas_call(
    _tc3_body,
    grid=(NBLKO,),
    in_specs=[
        pl.BlockSpec((BNO, HALF), lambda n: (n, 0)),
        pl.BlockSpec((BNO, HALF), lambda n: (n, 0)),
        pl.BlockSpec((BNO, 1), lambda n: (n, 0)),
        pl.BlockSpec((1, F), lambda n: (0, 0)),
    ],
    out_specs=pl.BlockSpec((BNO, F), lambda n: (n, 0)),
    out_shape=jax.ShapeDtypeStruct((N, F), jnp.float32),
)


def kernel(x, edge_index, W1, b1, W2, b2):
    src = edge_index[0]
    dst = edge_index[1]

    npad = EP - E
    padi = jnp.arange(npad, dtype=jnp.int32)
    src_p = jnp.concatenate([src, padi % N]).reshape(EROWS, SLOT)
    dst_p = jnp.concatenate([dst, N + padi % (NP - N)]).reshape(EROWS, SLOT)

    zeros1d = jnp.zeros((ZPT,), jnp.float32)
    zeros2d = jnp.zeros((ZPT, HALF), jnp.float32)
    ones_h = jnp.ones((SLOT,), jnp.float32)

    dinv_pad = _deg_dinv(dst_p, zeros1d, ones_h)
    x_pad = jnp.pad(x, ((0, NP - N), (0, 0)))
    dinv2d = dinv_pad.reshape(NP, 1)

    s1 = _tc1(x_pad, W1, dinv2d)
    a1_lo, a1_hi = _agg(s1, src_p, dst_p, zeros2d)
    s2 = _tc2(a1_lo, a1_hi, dinv2d, b1.reshape(1, F), W2)
    a2_lo, a2_hi = _agg(s2, src_p, dst_p, zeros2d)
    return _tc3(a2_lo, a2_hi, dinv2d, b2.reshape(1, F))


# speedup vs baseline: 13.4690x; 1.0036x over previous
import jax
import jax.numpy as jnp
from jax import lax
from jax.experimental import pallas as pl
from jax.experimental.pallas import tpu as pltpu
from jax.experimental.pallas import tpu_sc as plsc

N = 10000
E = 160000
F = 256
HALF = 128
NC = 2
NS = 16
NW = NC * NS

NP = 10240
EP = 163840
SLOT = 80
EROWS = EP // SLOT
RPT = EROWS // NS
ZPT = NP // NS
DPT = NP // NW
NB = 4
CH = RPT // 8

BN = 1024
NBLK = NP // BN
BNO = 1000
NBLKO = N // BNO

_mesh = plsc.VectorSubcoreMesh(core_axis_name="c", subcore_axis_name="s")


def _deg_dinv_body(dst2d, zeros1d, ones_h, dinv_out, acc, idxb, onesb, degb,
                   dinvb, hsem):
    c = lax.axis_index("c")
    s = lax.axis_index("s")
    wid = c * NS + s

    pltpu.sync_copy(zeros1d, acc.at[pl.ds(s * ZPT, ZPT)])
    pltpu.sync_copy(ones_h, onesb)
    pltpu.sync_copy(dst2d.at[pl.ds(s * RPT, RPT)], idxb)
    plsc.subcore_barrier()

    @pl.loop(0, RPT, step=16)
    def _(g0):
        for u in range(16):
            pltpu.async_copy(onesb, acc.at[idxb.at[g0 + u]], hsem, add=True)
        for u in range(16):
            pltpu.make_async_copy(onesb, acc.at[pl.ds(0, SLOT)], hsem).wait()

    plsc.subcore_barrier()

    pltpu.sync_copy(acc.at[pl.ds(wid * DPT, DPT)], degb)

    @pl.loop(0, DPT // 16)
    def _(j):
        d = jnp.maximum(degb[pl.ds(j * 16, 16)], 1.0)
        i = lax.bitcast_convert_type(d, jnp.int32)
        y = lax.bitcast_convert_type(jnp.int32(0x5F3759DF) - (i >> 1), jnp.float32)
        y = y * (1.5 - 0.5 * d * y * y)
        y = y * (1.5 - 0.5 * d * y * y)
        y = y * (1.5 - 0.5 * d * y * y)
        dinvb[pl.ds(j * 16, 16)] = y

    pltpu.sync_copy(dinvb, dinv_out.at[pl.ds(wid * DPT, DPT)])


_deg_dinv = pl.kernel(
    _deg_dinv_body,
    out_type=jax.ShapeDtypeStruct((NP,), jnp.float32),
    mesh=_mesh,
    scratch_types=[
        pltpu.VMEM_SHARED((NP,), jnp.float32),
        pltpu.VMEM((RPT, SLOT), jnp.int32),
        pltpu.VMEM((SLOT,), jnp.float32),
        pltpu.VMEM((DPT,), jnp.float32),
        pltpu.VMEM((DPT,), jnp.float32),
        pltpu.SemaphoreType.DMA,
    ],
)


def _agg_body(S, src2d, dst2d, zeros2d, out_lo, out_hi, acc, srcb, dstb,
              rb0, rb1, rb2, rb3, gsem, ssem):
    c = lax.axis_index("c")
    s = lax.axis_index("s")
    bufs = [rb0, rb1, rb2, rb3]

    def wait_scatter(b):
        pltpu.make_async_copy(bufs[b], acc.at[pl.ds(0, SLOT)], ssem).wait()

    pltpu.sync_copy(zeros2d, acc.at[pl.ds(s * ZPT, ZPT)])

    off = c * NP

    plsc.subcore_barrier()

    @pl.loop(0, RPT // CH)
    def _(ch):
        base = s * RPT + ch * CH
        pltpu.sync_copy(src2d.at[pl.ds(base, CH)], srcb)
        pltpu.sync_copy(dst2d.at[pl.ds(base, CH)], dstb)

        @pl.loop(0, CH)
        def _(i):
            for j in range(SLOT // 16):
                srcb[i, pl.ds(j * 16, 16)] = srcb[i, pl.ds(j * 16, 16)] + off

        pltpu.async_copy(S.at[srcb.at[0]], bufs[0], gsem)
        pltpu.async_copy(S.at[srcb.at[1]], bufs[1], gsem)

        @pl.loop(0, CH // NB)
        def _(t):
            for b in range(NB):
                r = t * NB + b
                pltpu.make_async_copy(S.at[srcb.at[r]], bufs[b], gsem).wait()
                pltpu.async_copy(bufs[b], acc.at[dstb.at[r]], ssem, add=True)

                @pl.when(r >= 2)
                def _():
                    wait_scatter((b + 2) % NB)

                @pl.when(r + 2 < CH)
                def _():
                    pltpu.async_copy(
                        S.at[srcb.at[r + 2]], bufs[(b + 2) % NB], gsem
                    )

        wait_scatter(NB - 2)
        wait_scatter(NB - 1)

    plsc.subcore_barrier()

    @pl.when(c == 0)
    def _():
        pltpu.sync_copy(acc.at[pl.ds(s * ZPT, ZPT)], out_lo.at[pl.ds(s * ZPT, ZPT)])

    @pl.when(c == 1)
    def _():
        pltpu.sync_copy(acc.at[pl.ds(s * ZPT, ZPT)], out_hi.at[pl.ds(s * ZPT, ZPT)])


_agg = pl.kernel(
    _agg_body,
    out_type=(
        jax.ShapeDtypeStruct((NP, HALF), jnp.float32),
        jax.ShapeDtypeStruct((NP, HALF), jnp.float32),
    ),
    mesh=_mesh,
    scratch_types=[
        pltpu.VMEM_SHARED((NP, HALF), jnp.float32),
        pltpu.VMEM((CH, SLOT), jnp.int32),
        pltpu.VMEM((CH, SLOT), jnp.int32),
        pltpu.VMEM((SLOT, HALF), jnp.float32),
        pltpu.VMEM((SLOT, HALF), jnp.float32),
        pltpu.VMEM((SLOT, HALF), jnp.float32),
        pltpu.VMEM((SLOT, HALF), jnp.float32),
        pltpu.SemaphoreType.DMA,
        pltpu.SemaphoreType.DMA,
    ],
)


def _tc1_body(x_ref, w_ref, dinv_ref, o_ref):
    o_ref[...] = (
        jnp.dot(
            x_ref[...].astype(jnp.bfloat16),
            w_ref[...],
            preferred_element_type=jnp.float32,
        )
        * dinv_ref[...]
    )


_tc1 = pl.pallas_call(
    _tc1_body,
    grid=(NBLK, 2),
    in_specs=[
        pl.BlockSpec((BN, F), lambda n, h: (n, 0)),
        pl.BlockSpec((F, HALF), lambda n, h: (0, h)),
        pl.BlockSpec((BN, 1), lambda n, h: (n, 0)),
    ],
    out_specs=pl.BlockSpec((BN, HALF), lambda n, h: (h * NBLK + n, 0)),
    out_shape=jax.ShapeDtypeStruct((2 * NP, HALF), jnp.float32),
)


def _tc2_body(alo_ref, ahi_ref, dinv_ref, b1_ref, w2_ref, o_ref):
    a = jnp.concatenate([alo_ref[...], ahi_ref[...]], axis=1)
    hid = jnp.maximum(a * dinv_ref[...] + b1_ref[...], 0.0)
    o_ref[...] = (
        jnp.dot(
            hid.astype(jnp.bfloat16),
            w2_ref[...],
            preferred_element_type=jnp.float32,
        )
        * dinv_ref[...]
    )


_tc2 = pl.pallas_call(
    _tc2_body,
    grid=(NBLK, 2),
    in_specs=[
        pl.BlockSpec((BN, HALF), lambda n, h: (n, 0)),
        pl.BlockSpec((BN, HALF), lambda n, h: (n, 0)),
        pl.BlockSpec((BN, 1), lambda n, h: (n, 0)),
        pl.BlockSpec((1, F), lambda n, h: (0, 0)),
        pl.BlockSpec((F, HALF), lambda n, h: (0, h)),
    ],
    out_specs=pl.BlockSpec((BN, HALF), lambda n, h: (h * NBLK + n, 0)),
    out_shape=jax.ShapeDtypeStruct((2 * NP, HALF), jnp.float32),
)


def _tc3_body(alo_ref, ahi_ref, dinv_ref, b2_ref, o_ref):
    a = jnp.concatenate([alo_ref[...], ahi_ref[...]], axis=1)
    o_ref[...] = a * dinv_ref[...] + b2_ref[...]


_tc3 = pl.pallas_call(
    _tc3_body,
    grid=(NBLKO,),
    in_specs=[
        pl.BlockSpec((BNO, HALF), lambda n: (n, 0)),
        pl.BlockSpec((BNO, HALF), lambda n: (n, 0)),
        pl.BlockSpec((BNO, 1), lambda n: (n, 0)),
        pl.BlockSpec((1, F), lambda n: (0, 0)),
    ],
    out_specs=pl.BlockSpec((BNO, F), lambda n: (n, 0)),
    out_shape=jax.ShapeDtypeStruct((N, F), jnp.float32),
)


def kernel(x, edge_index, W1, b1, W2, b2):
    src = edge_index[0]
    dst = edge_index[1]

    npad = EP - E
    padi = jnp.arange(npad, dtype=jnp.int32)
    src_p = jnp.concatenate([src, padi % N]).reshape(EROWS, SLOT)
    dst_p = jnp.concatenate([dst, N + padi % (NP - N)]).reshape(EROWS, SLOT)

    zeros1d = jnp.zeros((ZPT,), jnp.float32)
    zeros2d = jnp.zeros((ZPT, HALF), jnp.float32)
    ones_h = jnp.ones((SLOT,), jnp.float32)

    dinv_pad = _deg_dinv(dst_p, zeros1d, ones_h)
    dinv2d = dinv_pad.reshape(NP, 1)

    s1 = _tc1(x, W1.astype(jnp.bfloat16), dinv2d)
    a1_lo, a1_hi = _agg(s1, src_p, dst_p, zeros2d)
    s2 = _tc2(a1_lo, a1_hi, dinv2d, b1.reshape(1, F), W2.astype(jnp.bfloat16))
    a2_lo, a2_hi = _agg(s2, src_p, dst_p, zeros2d)
    return _tc3(a2_lo, a2_hi, dinv2d, b2.reshape(1, F))


# speedup vs baseline: 13.9274x; 1.0340x over previous
import jax
import jax.numpy as jnp
import numpy as np
from jax import lax
from jax.experimental import pallas as pl
from jax.experimental.pallas import tpu as pltpu
from jax.experimental.pallas import tpu_sc as plsc

N = 10000
E = 160000
F = 256
HALF = 128
NC = 2
NS = 16
NW = NC * NS

NP = 10240
EP = 163840
SLOT = 80
EROWS = EP // SLOT
RPT = EROWS // NS
ZPT = NP // NS
DPT = NP // NW
NB = 4
CH = RPT // 8

BN = 2048
NBLK = NP // BN
BNO = 1000
NBLKO = N // BNO

_mesh = plsc.VectorSubcoreMesh(core_axis_name="c", subcore_axis_name="s")

_PAD_SRC = jnp.asarray(np.arange(EP - E) % N, dtype=jnp.int32)
_PAD_DST = jnp.asarray(N + np.arange(EP - E) % (NP - N), dtype=jnp.int32)


def _deg_dinv_body(dst2d, zeros1d, ones_h, dinv_out, acc, idxb, onesb, degb,
                   dinvb, hsem):
    c = lax.axis_index("c")
    s = lax.axis_index("s")
    wid = c * NS + s

    pltpu.sync_copy(zeros1d, acc.at[pl.ds(s * ZPT, ZPT)])
    pltpu.sync_copy(ones_h, onesb)
    pltpu.sync_copy(dst2d.at[pl.ds(s * RPT, RPT)], idxb)
    plsc.subcore_barrier()

    @pl.loop(0, RPT, step=16)
    def _(g0):
        for u in range(16):
            pltpu.async_copy(onesb, acc.at[idxb.at[g0 + u]], hsem, add=True)
        for u in range(16):
            pltpu.make_async_copy(onesb, acc.at[pl.ds(0, SLOT)], hsem).wait()

    plsc.subcore_barrier()

    pltpu.sync_copy(acc.at[pl.ds(wid * DPT, DPT)], degb)

    @pl.loop(0, DPT // 16)
    def _(j):
        d = jnp.maximum(degb[pl.ds(j * 16, 16)], 1.0)
        i = lax.bitcast_convert_type(d, jnp.int32)
        y = lax.bitcast_convert_type(jnp.int32(0x5F3759DF) - (i >> 1), jnp.float32)
        y = y * (1.5 - 0.5 * d * y * y)
        y = y * (1.5 - 0.5 * d * y * y)
        y = y * (1.5 - 0.5 * d * y * y)
        dinvb[pl.ds(j * 16, 16)] = y

    pltpu.sync_copy(dinvb, dinv_out.at[pl.ds(wid * DPT, DPT)])


_deg_dinv = pl.kernel(
    _deg_dinv_body,
    out_type=jax.ShapeDtypeStruct((NP,), jnp.float32),
    mesh=_mesh,
    scratch_types=[
        pltpu.VMEM_SHARED((NP,), jnp.float32),
        pltpu.VMEM((RPT, SLOT), jnp.int32),
        pltpu.VMEM((SLOT,), jnp.float32),
        pltpu.VMEM((DPT,), jnp.float32),
        pltpu.VMEM((DPT,), jnp.float32),
        pltpu.SemaphoreType.DMA,
    ],
)


def _agg_body(S, src2d, dst2d, zeros2d, out_lo, out_hi, acc, srcb, dstb,
              rb0, rb1, rb2, rb3, gsem, ssem):
    c = lax.axis_index("c")
    s = lax.axis_index("s")
    bufs = [rb0, rb1, rb2, rb3]

    def wait_scatter(b):
        pltpu.make_async_copy(bufs[b], acc.at[pl.ds(0, SLOT)], ssem).wait()

    pltpu.sync_copy(zeros2d, acc.at[pl.ds(s * ZPT, ZPT)])

    off = c * NP

    plsc.subcore_barrier()

    @pl.loop(0, RPT // CH)
    def _(ch):
        base = s * RPT + ch * CH
        pltpu.sync_copy(src2d.at[pl.ds(base, CH)], srcb)
        pltpu.sync_copy(dst2d.at[pl.ds(base, CH)], dstb)

        @pl.loop(0, CH)
        def _(i):
            for j in range(SLOT // 16):
                srcb[i, pl.ds(j * 16, 16)] = srcb[i, pl.ds(j * 16, 16)] + off

        pltpu.async_copy(S.at[srcb.at[0]], bufs[0], gsem)
        pltpu.async_copy(S.at[srcb.at[1]], bufs[1], gsem)

        @pl.loop(0, CH // NB)
        def _(t):
            for b in range(NB):
                r = t * NB + b
                pltpu.make_async_copy(S.at[srcb.at[r]], bufs[b], gsem).wait()
                pltpu.async_copy(bufs[b], acc.at[dstb.at[r]], ssem, add=True)

                @pl.when(r >= 2)
                def _():
                    wait_scatter((b + 2) % NB)

                @pl.when(r + 2 < CH)
                def _():
                    pltpu.async_copy(
                        S.at[srcb.at[r + 2]], bufs[(b + 2) % NB], gsem
                    )

        wait_scatter(NB - 2)
        wait_scatter(NB - 1)

    plsc.subcore_barrier()

    @pl.when(c == 0)
    def _():
        pltpu.sync_copy(acc.at[pl.ds(s * ZPT, ZPT)], out_lo.at[pl.ds(s * ZPT, ZPT)])

    @pl.when(c == 1)
    def _():
        pltpu.sync_copy(acc.at[pl.ds(s * ZPT, ZPT)], out_hi.at[pl.ds(s * ZPT, ZPT)])


_agg = pl.kernel(
    _agg_body,
    out_type=(
        jax.ShapeDtypeStruct((NP, HALF), jnp.float32),
        jax.ShapeDtypeStruct((NP, HALF), jnp.float32),
    ),
    mesh=_mesh,
    scratch_types=[
        pltpu.VMEM_SHARED((NP, HALF), jnp.float32),
        pltpu.VMEM((CH, SLOT), jnp.int32),
        pltpu.VMEM((CH, SLOT), jnp.int32),
        pltpu.VMEM((SLOT, HALF), jnp.float32),
        pltpu.VMEM((SLOT, HALF), jnp.float32),
        pltpu.VMEM((SLOT, HALF), jnp.float32),
        pltpu.VMEM((SLOT, HALF), jnp.float32),
        pltpu.SemaphoreType.DMA,
        pltpu.SemaphoreType.DMA,
    ],
)


def _tc1_body(x_ref, w_ref, dinv_ref, o_ref):
    o_ref[...] = (
        jnp.dot(
            x_ref[...].astype(jnp.bfloat16),
            w_ref[...],
            preferred_element_type=jnp.float32,
        )
        * dinv_ref[...]
    )


_tc1 = pl.pallas_call(
    _tc1_body,
    grid=(NBLK, 2),
    in_specs=[
        pl.BlockSpec((BN, F), lambda n, h: (n, 0)),
        pl.BlockSpec((F, HALF), lambda n, h: (0, h)),
        pl.BlockSpec((BN, 1), lambda n, h: (n, 0)),
    ],
    out_specs=pl.BlockSpec((BN, HALF), lambda n, h: (h * NBLK + n, 0)),
    out_shape=jax.ShapeDtypeStruct((2 * NP, HALF), jnp.float32),
)


def _tc2_body(alo_ref, ahi_ref, dinv_ref, b1_ref, w2_ref, o_ref):
    a = jnp.concatenate([alo_ref[...], ahi_ref[...]], axis=1)
    hid = jnp.maximum(a * dinv_ref[...] + b1_ref[...], 0.0)
    o_ref[...] = (
        jnp.dot(
            hid.astype(jnp.bfloat16),
            w2_ref[...],
            preferred_element_type=jnp.float32,
        )
        * dinv_ref[...]
    )


_tc2 = pl.pallas_call(
    _tc2_body,
    grid=(NBLK, 2),
    in_specs=[
        pl.BlockSpec((BN, HALF), lambda n, h: (n, 0)),
        pl.BlockSpec((BN, HALF), lambda n, h: (n, 0)),
        pl.BlockSpec((BN, 1), lambda n, h: (n, 0)),
        pl.BlockSpec((1, F), lambda n, h: (0, 0)),
        pl.BlockSpec((F, HALF), lambda n, h: (0, h)),
    ],
    out_specs=pl.BlockSpec((BN, HALF), lambda n, h: (h * NBLK + n, 0)),
    out_shape=jax.ShapeDtypeStruct((2 * NP, HALF), jnp.float32),
)


def _tc3_body(alo_ref, ahi_ref, dinv_ref, b2_ref, o_ref):
    a = jnp.concatenate([alo_ref[...], ahi_ref[...]], axis=1)
    o_ref[...] = a * dinv_ref[...] + b2_ref[...]


_tc3 = pl.pallas_call(
    _tc3_body,
    grid=(NBLKO,),
    in_specs=[
        pl.BlockSpec((BNO, HALF), lambda n: (n, 0)),
        pl.BlockSpec((BNO, HALF), lambda n: (n, 0)),
        pl.BlockSpec((BNO, 1), lambda n: (n, 0)),
        pl.BlockSpec((1, F), lambda n: (0, 0)),
    ],
    out_specs=pl.BlockSpec((BNO, F), lambda n: (n, 0)),
    out_shape=jax.ShapeDtypeStruct((N, F), jnp.float32),
)


def kernel(x, edge_index, W1, b1, W2, b2):
    src = edge_index[0]
    dst = edge_index[1]

    src_p = jnp.concatenate([src, _PAD_SRC]).reshape(EROWS, SLOT)
    dst_p = jnp.concatenate([dst, _PAD_DST]).reshape(EROWS, SLOT)

    zeros1d = jnp.zeros((ZPT,), jnp.float32)
    zeros2d = jnp.zeros((ZPT, HALF), jnp.float32)
    ones_h = jnp.ones((SLOT,), jnp.float32)

    dinv_pad = _deg_dinv(dst_p, zeros1d, ones_h)
    dinv2d = dinv_pad.reshape(NP, 1)

    s1 = _tc1(x, W1.astype(jnp.bfloat16), dinv2d)
    a1_lo, a1_hi = _agg(s1, src_p, dst_p, zeros2d)
    s2 = _tc2(a1_lo, a1_hi, dinv2d, b1.reshape(1, F), W2.astype(jnp.bfloat16))
    a2_lo, a2_hi = _agg(s2, src_p, dst_p, zeros2d)
    return _tc3(a2_lo, a2_hi, dinv2d, b2.reshape(1, F))


# speedup vs baseline: 14.0409x; 1.0081x over previous
import jax
import jax.numpy as jnp
import numpy as np
from jax import lax
from jax.experimental import pallas as pl
from jax.experimental.pallas import tpu as pltpu
from jax.experimental.pallas import tpu_sc as plsc

N = 10000
E = 160000
F = 256
HALF = 128
NC = 2
NS = 16
NW = NC * NS

NP = 10240
EP = 163840
SLOT = 80
EROWS = EP // SLOT
RPT = EROWS // NS
ZPT = NP // NS
DPT = NP // NW
NB = 4
CH = RPT // 8

BN = 2560
NBLK = NP // BN
BNO = 1000
NBLKO = N // BNO

_mesh = plsc.VectorSubcoreMesh(core_axis_name="c", subcore_axis_name="s")

_PAD_SRC = jnp.asarray(np.arange(EP - E) % N, dtype=jnp.int32)
_PAD_DST = jnp.asarray(N + np.arange(EP - E) % (NP - N), dtype=jnp.int32)


def _deg_dinv_body(dst2d, zeros1d, ones_h, dinv_out, acc, idxb, onesb, degb,
                   dinvb, hsem):
    c = lax.axis_index("c")
    s = lax.axis_index("s")
    wid = c * NS + s

    pltpu.sync_copy(zeros1d, acc.at[pl.ds(s * ZPT, ZPT)])
    pltpu.sync_copy(ones_h, onesb)
    pltpu.sync_copy(dst2d.at[pl.ds(s * RPT, RPT)], idxb)
    plsc.subcore_barrier()

    @pl.loop(0, RPT, step=16)
    def _(g0):
        for u in range(16):
            pltpu.async_copy(onesb, acc.at[idxb.at[g0 + u]], hsem, add=True)
        for u in range(16):
            pltpu.make_async_copy(onesb, acc.at[pl.ds(0, SLOT)], hsem).wait()

    plsc.subcore_barrier()

    pltpu.sync_copy(acc.at[pl.ds(wid * DPT, DPT)], degb)

    @pl.loop(0, DPT // 16)
    def _(j):
        d = jnp.maximum(degb[pl.ds(j * 16, 16)], 1.0)
        i = lax.bitcast_convert_type(d, jnp.int32)
        y = lax.bitcast_convert_type(jnp.int32(0x5F3759DF) - (i >> 1), jnp.float32)
        y = y * (1.5 - 0.5 * d * y * y)
        y = y * (1.5 - 0.5 * d * y * y)
        y = y * (1.5 - 0.5 * d * y * y)
        dinvb[pl.ds(j * 16, 16)] = y

    pltpu.sync_copy(dinvb, dinv_out.at[pl.ds(wid * DPT, DPT)])


_deg_dinv = pl.kernel(
    _deg_dinv_body,
    out_type=jax.ShapeDtypeStruct((NP,), jnp.float32),
    mesh=_mesh,
    scratch_types=[
        pltpu.VMEM_SHARED((NP,), jnp.float32),
        pltpu.VMEM((RPT, SLOT), jnp.int32),
        pltpu.VMEM((SLOT,), jnp.float32),
        pltpu.VMEM((DPT,), jnp.float32),
        pltpu.VMEM((DPT,), jnp.float32),
        pltpu.SemaphoreType.DMA,
    ],
)


def _agg_body(S, src2d, dst2d, zeros2d, out_lo, out_hi, acc, srcb, dstb,
              rb0, rb1, rb2, rb3, gsem, ssem):
    c = lax.axis_index("c")
    s = lax.axis_index("s")
    bufs = [rb0, rb1, rb2, rb3]

    def wait_scatter(b):
        pltpu.make_async_copy(bufs[b], acc.at[pl.ds(0, SLOT)], ssem).wait()

    pltpu.sync_copy(zeros2d, acc.at[pl.ds(s * ZPT, ZPT)])

    off = c * NP

    plsc.subcore_barrier()

    @pl.loop(0, RPT // CH)
    def _(ch):
        base = s * RPT + ch * CH
        pltpu.sync_copy(src2d.at[pl.ds(base, CH)], srcb)
        pltpu.sync_copy(dst2d.at[pl.ds(base, CH)], dstb)

        @pl.loop(0, CH)
        def _(i):
            for j in range(SLOT // 16):
                srcb[i, pl.ds(j * 16, 16)] = srcb[i, pl.ds(j * 16, 16)] + off

        pltpu.async_copy(S.at[srcb.at[0]], bufs[0], gsem)
        pltpu.async_copy(S.at[srcb.at[1]], bufs[1], gsem)

        @pl.loop(0, CH // NB)
        def _(t):
            for b in range(NB):
                r = t * NB + b
                pltpu.make_async_copy(S.at[srcb.at[r]], bufs[b], gsem).wait()
                pltpu.async_copy(bufs[b], acc.at[dstb.at[r]], ssem, add=True)

                @pl.when(r >= 2)
                def _():
                    wait_scatter((b + 2) % NB)

                @pl.when(r + 2 < CH)
                def _():
                    pltpu.async_copy(
                        S.at[srcb.at[r + 2]], bufs[(b + 2) % NB], gsem
                    )

        wait_scatter(NB - 2)
        wait_scatter(NB - 1)

    plsc.subcore_barrier()

    @pl.when(c == 0)
    def _():
        pltpu.sync_copy(acc.at[pl.ds(s * ZPT, ZPT)], out_lo.at[pl.ds(s * ZPT, ZPT)])

    @pl.when(c == 1)
    def _():
        pltpu.sync_copy(acc.at[pl.ds(s * ZPT, ZPT)], out_hi.at[pl.ds(s * ZPT, ZPT)])


_agg = pl.kernel(
    _agg_body,
    out_type=(
        jax.ShapeDtypeStruct((NP, HALF), jnp.float32),
        jax.ShapeDtypeStruct((NP, HALF), jnp.float32),
    ),
    mesh=_mesh,
    scratch_types=[
        pltpu.VMEM_SHARED((NP, HALF), jnp.float32),
        pltpu.VMEM((CH, SLOT), jnp.int32),
        pltpu.VMEM((CH, SLOT), jnp.int32),
        pltpu.VMEM((SLOT, HALF), jnp.float32),
        pltpu.VMEM((SLOT, HALF), jnp.float32),
        pltpu.VMEM((SLOT, HALF), jnp.float32),
        pltpu.VMEM((SLOT, HALF), jnp.float32),
        pltpu.SemaphoreType.DMA,
        pltpu.SemaphoreType.DMA,
    ],
)


def _tc1_body(x_ref, w_ref, dinv_ref, o_ref):
    o_ref[...] = (
        jnp.dot(
            x_ref[...].astype(jnp.bfloat16),
            w_ref[...],
            preferred_element_type=jnp.float32,
        )
        * dinv_ref[...]
    )


_tc1 = pl.pallas_call(
    _tc1_body,
    grid=(NBLK, 2),
    in_specs=[
        pl.BlockSpec((BN, F), lambda n, h: (n, 0)),
        pl.BlockSpec((F, HALF), lambda n, h: (0, h)),
        pl.BlockSpec((BN, 1), lambda n, h: (n, 0)),
    ],
    out_specs=pl.BlockSpec((BN, HALF), lambda n, h: (h * NBLK + n, 0)),
    out_shape=jax.ShapeDtypeStruct((2 * NP, HALF), jnp.float32),
)


def _tc2_body(alo_ref, ahi_ref, dinv_ref, b1_ref, w2_ref, o_ref):
    a = jnp.concatenate([alo_ref[...], ahi_ref[...]], axis=1)
    hid = jnp.maximum(a * dinv_ref[...] + b1_ref[...], 0.0)
    o_ref[...] = (
        jnp.dot(
            hid.astype(jnp.bfloat16),
            w2_ref[...],
            preferred_element_type=jnp.float32,
        )
        * dinv_ref[...]
    )


_tc2 = pl.pallas_call(
    _tc2_body,
    grid=(NBLK, 2),
    in_specs=[
        pl.BlockSpec((BN, HALF), lambda n, h: (n, 0)),
        pl.BlockSpec((BN, HALF), lambda n, h: (n, 0)),
        pl.BlockSpec((BN, 1), lambda n, h: (n, 0)),
        pl.BlockSpec((1, F), lambda n, h: (0, 0)),
        pl.BlockSpec((F, HALF), lambda n, h: (0, h)),
    ],
    out_specs=pl.BlockSpec((BN, HALF), lambda n, h: (h * NBLK + n, 0)),
    out_shape=jax.ShapeDtypeStruct((2 * NP, HALF), jnp.float32),
)


def _tc3_body(alo_ref, ahi_ref, dinv_ref, b2_ref, o_ref):
    a = jnp.concatenate([alo_ref[...], ahi_ref[...]], axis=1)
    o_ref[...] = a * dinv_ref[...] + b2_ref[...]


_tc3 = pl.pallas_call(
    _tc3_body,
    grid=(NBLKO,),
    in_specs=[
        pl.BlockSpec((BNO, HALF), lambda n: (n, 0)),
        pl.BlockSpec((BNO, HALF), lambda n: (n, 0)),
        pl.BlockSpec((BNO, 1), lambda n: (n, 0)),
        pl.BlockSpec((1, F), lambda n: (0, 0)),
    ],
    out_specs=pl.BlockSpec((BNO, F), lambda n: (n, 0)),
    out_shape=jax.ShapeDtypeStruct((N, F), jnp.float32),
)


def kernel(x, edge_index, W1, b1, W2, b2):
    src = edge_index[0]
    dst = edge_index[1]

    src_p = jnp.concatenate([src, _PAD_SRC]).reshape(EROWS, SLOT)
    dst_p = jnp.concatenate([dst, _PAD_DST]).reshape(EROWS, SLOT)

    zeros1d = jnp.zeros((ZPT,), jnp.float32)
    zeros2d = jnp.zeros((ZPT, HALF), jnp.float32)
    ones_h = jnp.ones((SLOT,), jnp.float32)

    dinv_pad = _deg_dinv(dst_p, zeros1d, ones_h)
    dinv2d = dinv_pad.reshape(NP, 1)

    s1 = _tc1(x, W1.astype(jnp.bfloat16), dinv2d)
    a1_lo, a1_hi = _agg(s1, src_p, dst_p, zeros2d)
    s2 = _tc2(a1_lo, a1_hi, dinv2d, b1.reshape(1, F), W2.astype(jnp.bfloat16))
    a2_lo, a2_hi = _agg(s2, src_p, dst_p, zeros2d)
    return _tc3(a2_lo, a2_hi, dinv2d, b2.reshape(1, F))
